# Initial kernel scaffold; baseline (speedup 1.0000x reference)
import jax
import jax.numpy as jnp
from jax.experimental import pallas as pl


def kernel(x, edge_index, W1, b1, W2, b2, W3, b3, Wl, bl, Wi, bi, Wg, bg):
    raise NotImplementedError("write your pallas kernel here")


# speedup vs baseline: 12.0090x; 12.0090x over previous
import functools

import jax
import jax.numpy as jnp
from jax import lax
from jax.experimental import pallas as pl
from jax.experimental.pallas import tpu as pltpu
from jax.experimental.pallas import tpu_sc as plsc

N = 10000
E = 320000
D_IN = 128
D_H = 256
D_OUT = 128

NP = 10240
NE = 323584
CH = 128
NTILES = 16
RPT = NP // NTILES

F_S = 8


def _make_spmm_es():
    F = D_IN
    EPT = NE // (2 * NTILES)
    NCH = EPT // CH
    mesh = plsc.VectorSubcoreMesh(core_axis_name="c", subcore_axis_name="s")

    @functools.partial(
        pl.kernel,
        mesh=mesh,
        out_type=jax.ShapeDtypeStruct((2 * NP, F), jnp.float32),
        scratch_types=[
            pltpu.VMEM_SHARED((NP, F), jnp.float32),
            pltpu.VMEM((CH,), jnp.int32),
            pltpu.VMEM((CH,), jnp.int32),
            pltpu.VMEM((CH, F), jnp.float32),
            pltpu.SemaphoreType.DMA,
        ],
    )
    def spmm(tab, src_i, dst_i, zrows, out, acc, sidx, didx, rows, sem):
        c = lax.axis_index("c")
        s = lax.axis_index("s")
        pltpu.sync_copy(zrows, acc.at[pl.ds(s * RPT, RPT)])
        plsc.subcore_barrier()
        base = (c * NTILES + s) * EPT

        def chunk(j, carry):
            off = base + j * CH
            pltpu.sync_copy(src_i.at[pl.ds(off, CH)], sidx)
            pltpu.sync_copy(dst_i.at[pl.ds(off, CH)], didx)
            pltpu.async_copy(tab.at[sidx], rows, sem).wait()
            pltpu.sync_copy(rows, acc.at[didx], add=True)
            return carry

        lax.fori_loop(0, NCH, chunk, 0)
        plsc.subcore_barrier()
        pltpu.sync_copy(acc.at[pl.ds(s * RPT, RPT)],
                        out.at[pl.ds(c * NP + s * RPT, RPT)])

    return spmm


def _make_spmm_fs():
    Fh = D_H // 2
    EPT = NE // NTILES
    NCH = EPT // CH
    mesh = plsc.VectorSubcoreMesh(core_axis_name="c", subcore_axis_name="s")

    @functools.partial(
        pl.kernel,
        mesh=mesh,
        out_type=jax.ShapeDtypeStruct((2 * NP, Fh), jnp.float32),
        scratch_types=[
            pltpu.VMEM_SHARED((NP, Fh), jnp.float32),
            pltpu.VMEM((CH,), jnp.int32),
            pltpu.VMEM((CH,), jnp.int32),
            pltpu.VMEM((CH, Fh), jnp.float32),
            pltpu.SemaphoreType.DMA,
        ],
    )
    def spmm(tab, src_i, dst_i, zrows, out, acc, sidx, didx, rows, sem):
        c = lax.axis_index("c")
        s = lax.axis_index("s")
        pltpu.sync_copy(zrows, acc.at[pl.ds(s * RPT, RPT)])
        plsc.subcore_barrier()
        base = s * EPT

        def chunk(j, carry):
            off = base + j * CH
            pltpu.sync_copy(src_i.at[pl.ds(off, CH)], sidx)
            pltpu.sync_copy(dst_i.at[pl.ds(off, CH)], didx)
            for k in range(CH // 16):
                sl = pl.ds(k * 16, 16)
                sidx[sl] = sidx[sl] * 2 + c
            pltpu.async_copy(tab.at[sidx], rows, sem).wait()
            pltpu.sync_copy(rows, acc.at[didx], add=True)
            return carry

        lax.fori_loop(0, NCH, chunk, 0)
        plsc.subcore_barrier()
        pltpu.sync_copy(acc.at[pl.ds(s * RPT, RPT)],
                        out.at[pl.ds(c * NP + s * RPT, RPT)])

    return spmm


NR = NP // 128


def _make_deg():
    EPT = NE // (2 * NTILES)
    mesh = plsc.VectorSubcoreMesh(core_axis_name="c", subcore_axis_name="s")

    @functools.partial(
        pl.kernel,
        mesh=mesh,
        compiler_params=pltpu.CompilerParams(needs_layout_passes=False),
        out_type=jax.ShapeDtypeStruct((2 * NR, 128), jnp.float32),
        scratch_types=[
            pltpu.VMEM_SHARED((NR, 128), jnp.float32),
            pltpu.VMEM((NR, 128), jnp.float32),
            pltpu.VMEM((EPT,), jnp.int32),
            pltpu.VMEM((NR,), jnp.int32),
        ],
    )
    def degk(dst_i, zrows, out, acc, loc, didx, rid):
        c = lax.axis_index("c")
        s = lax.axis_index("s")
        pltpu.sync_copy(zrows, acc.at[pl.ds(0, NR)])
        pltpu.sync_copy(zrows, loc)
        base = (c * NTILES + s) * EPT
        pltpu.sync_copy(dst_i.at[pl.ds(base, EPT)], didx)
        for k in range(NR // 16):
            rid[pl.ds(k * 16, 16)] = lax.iota(jnp.int32, 16) + (k * 16)
        plsc.subcore_barrier()
        ones16 = jnp.full((16,), 1.0, jnp.float32)

        def step(j, carry):
            d16 = didx[pl.ds(j * 16, 16)]
            plsc.addupdate_scatter(
                loc,
                [lax.shift_right_logical(d16, 7), lax.bitwise_and(d16, 127)],
                ones16)
            return carry

        lax.fori_loop(0, EPT // 16, step, 0)
        pltpu.sync_copy(loc, acc.at[rid], add=True)
        plsc.subcore_barrier()

        @pl.when(s < NR // 8)
        def _():
            pltpu.sync_copy(acc.at[pl.ds(s * 8, 8)],
                            out.at[pl.ds(c * NR + s * 8, 8)])

    return degk


def _make_cseg():
    EPT = NE // (2 * NTILES)
    mesh = plsc.VectorSubcoreMesh(core_axis_name="c", subcore_axis_name="s")

    @functools.partial(
        pl.kernel,
        mesh=mesh,
        compiler_params=pltpu.CompilerParams(needs_layout_passes=False),
        out_type=jax.ShapeDtypeStruct((2 * NR, 128), jnp.float32),
        scratch_types=[
            pltpu.VMEM_SHARED((NR, 128), jnp.float32),
            pltpu.VMEM((NR, 128), jnp.float32),
            pltpu.VMEM((NR, 128), jnp.float32),
            pltpu.VMEM((EPT,), jnp.int32),
            pltpu.VMEM((EPT,), jnp.int32),
            pltpu.VMEM((NR,), jnp.int32),
        ],
    )
    def cseg(dinv_h, src_i, dst_i, zrows, out, acc, dinv_v, loc,
             sidx, didx, rid):
        c = lax.axis_index("c")
        s = lax.axis_index("s")
        pltpu.sync_copy(zrows, acc.at[pl.ds(0, NR)])
        pltpu.sync_copy(zrows, loc)
        pltpu.sync_copy(dinv_h, dinv_v)
        base = (c * NTILES + s) * EPT
        pltpu.sync_copy(src_i.at[pl.ds(base, EPT)], sidx)
        pltpu.sync_copy(dst_i.at[pl.ds(base, EPT)], didx)
        for k in range(NR // 16):
            rid[pl.ds(k * 16, 16)] = lax.iota(jnp.int32, 16) + (k * 16)
        plsc.subcore_barrier()

        def step(j, carry):
            d16 = didx[pl.ds(j * 16, 16)]
            vals = plsc.load_gather(
                dinv_v,
                [lax.shift_right_logical(d16, 7),
                 lax.bitwise_and(d16, 127)])
            s16 = sidx[pl.ds(j * 16, 16)]
            plsc.addupdate_scatter(
                loc,
                [lax.shift_right_logical(s16, 7), lax.bitwise_and(s16, 127)],
                vals)
            return carry

        lax.fori_loop(0, EPT // 16, step, 0)
        pltpu.sync_copy(loc, acc.at[rid], add=True)
        plsc.subcore_barrier()

        @pl.when(s < NR // 8)
        def _():
            pltpu.sync_copy(acc.at[pl.ds(s * 8, 8)],
                            out.at[pl.ds(c * NR + s * 8, 8)])

    return cseg


_spmm128 = _make_spmm_es()
_spmm256 = _make_spmm_fs()
_degk = _make_deg()
_cseg = _make_cseg()


_BM = 1024


def _tc0_body(x_ref, dega_ref, degb_ref, mask_ref,
              xs_ref, dinv_ref, dinv1_ref):
    deg = dega_ref[...] + degb_ref[...] + 1.0
    dinv = mask_ref[...] * jax.lax.rsqrt(deg)
    xs_ref[...] = x_ref[...] * dinv
    dinv_ref[...] = dinv
    dinv1_ref[...] = dinv[:, 0]


def _tc0(x, dega, degb, mask):
    grid = (NP // _BM,)
    return pl.pallas_call(
        _tc0_body,
        grid=grid,
        in_specs=[
            pl.BlockSpec((_BM, D_IN), lambda i: (i, 0)),
            pl.BlockSpec((_BM, 1), lambda i: (i, 0)),
            pl.BlockSpec((_BM, 1), lambda i: (i, 0)),
            pl.BlockSpec((_BM, 1), lambda i: (i, 0)),
        ],
        out_specs=[
            pl.BlockSpec((_BM, D_IN), lambda i: (i, 0)),
            pl.BlockSpec((_BM, 1), lambda i: (i, 0)),
            pl.BlockSpec((_BM,), lambda i: (i,)),
        ],
        out_shape=[
            jax.ShapeDtypeStruct((NP, D_IN), jnp.float32),
            jax.ShapeDtypeStruct((NP, 1), jnp.float32),
            jax.ShapeDtypeStruct((NP,), jnp.float32),
        ],
    )(x, dega, degb, mask)


def _tc1_body(acc_a, acc_b, xs_ref, dinv_ref, w_ref, b_ref, out_ref):
    acc = acc_a[...] + acc_b[...]
    dinv = dinv_ref[...]
    agg = (acc + xs_ref[...]) * dinv
    h = jnp.maximum(jnp.dot(agg, w_ref[...],
                            preferred_element_type=jnp.float32) + b_ref[...],
                    0.0)
    out_ref[...] = h * dinv


def _tc1(acc_a, acc_b, xs, dinv, W1, b1):
    grid = (NP // _BM,)
    return pl.pallas_call(
        _tc1_body,
        grid=grid,
        in_specs=[
            pl.BlockSpec((_BM, D_IN), lambda i: (i, 0)),
            pl.BlockSpec((_BM, D_IN), lambda i: (i, 0)),
            pl.BlockSpec((_BM, D_IN), lambda i: (i, 0)),
            pl.BlockSpec((_BM, 1), lambda i: (i, 0)),
            pl.BlockSpec((D_IN, D_H), lambda i: (0, 0)),
            pl.BlockSpec((1, D_H), lambda i: (0, 0)),
        ],
        out_specs=pl.BlockSpec((_BM, D_H), lambda i: (i, 0)),
        out_shape=jax.ShapeDtypeStruct((NP, D_H), jnp.float32),
    )(acc_a, acc_b, xs, dinv, W1, b1)


def _tc2_body(acc_lo, acc_hi, hs_ref, dinv_ref, ca_ref, cb_ref,
              w_ref, b_ref, u_ref):
    i = pl.program_id(0)
    acc = jnp.concatenate([acc_lo[...], acc_hi[...]], axis=1)
    dinv = dinv_ref[...]
    agg = (acc + hs_ref[...]) * dinv
    h2 = jnp.maximum(jnp.dot(agg, w_ref[...],
                             preferred_element_type=jnp.float32) + b_ref[...],
                     0.0)
    v = dinv * (ca_ref[...] + cb_ref[...] + dinv)
    part = jnp.sum(v * h2, axis=0, keepdims=True)

    @pl.when(i == 0)
    def _():
        u_ref[...] = jnp.zeros_like(u_ref)

    u_ref[0:1, :] += part


def _tc2(acc_lo, acc_hi, hs1, dinv, ca, cb, W2, b2):
    grid = (NP // _BM,)
    return pl.pallas_call(
        _tc2_body,
        grid=grid,
        in_specs=[
            pl.BlockSpec((_BM, D_H // 2), lambda i: (i, 0)),
            pl.BlockSpec((_BM, D_H // 2), lambda i: (i, 0)),
            pl.BlockSpec((_BM, D_H), lambda i: (i, 0)),
            pl.BlockSpec((_BM, 1), lambda i: (i, 0)),
            pl.BlockSpec((_BM, 1), lambda i: (i, 0)),
            pl.BlockSpec((_BM, 1), lambda i: (i, 0)),
            pl.BlockSpec((D_H, D_H), lambda i: (0, 0)),
            pl.BlockSpec((1, D_H), lambda i: (0, 0)),
        ],
        out_specs=pl.BlockSpec((8, D_H), lambda i: (0, 0)),
        out_shape=jax.ShapeDtypeStruct((8, D_H), jnp.float32),
    )(acc_lo, acc_hi, hs1, dinv, ca, cb, W2, b2)


def _tc3_body(u_ref, w3_ref, b3_ref, wh_ref, bh_ref, emb_ref, heads_ref):
    emb = jnp.dot(u_ref[...] * (1.0 / N), w3_ref[...],
                  preferred_element_type=jnp.float32) + b3_ref[...]
    emb_ref[...] = emb
    z = jnp.dot(emb, wh_ref[...], preferred_element_type=jnp.float32) \
        + bh_ref[...]
    heads_ref[...] = 1.0 / (1.0 + jnp.exp(-z))


def _tc3(u, W3, b3, Wh, bh):
    return pl.pallas_call(
        _tc3_body,
        out_shape=[
            jax.ShapeDtypeStruct((1, D_OUT), jnp.float32),
            jax.ShapeDtypeStruct((1, 8), jnp.float32),
        ],
    )(u, W3, b3, Wh, bh)


def kernel(x, edge_index, W1, b1, W2, b2, W3, b3, Wl, bl, Wi, bi, Wg, bg):
    f32 = jnp.float32
    src = edge_index[0]
    dst = edge_index[1]
    pad = jnp.full((NE - E,), N, dtype=jnp.int32)
    src_p = jnp.concatenate([src, pad])
    dst_p = jnp.concatenate([dst, pad])

    x_p = jnp.zeros((NP, D_IN), f32).at[:N].set(x)
    mask = jnp.zeros((NP, 1), f32).at[:N].set(1.0)

    zrows_t = jnp.zeros((NR, 128), f32)

    deg2 = _degk(dst_p, zrows_t)
    dega = deg2[:NR].reshape(NP, 1)
    degb = deg2[NR:].reshape(NP, 1)

    xs, dinv, dinv1 = _tc0(x_p, dega, degb, mask)

    c2 = _cseg(dinv1.reshape(NR, 128), src_p, dst_p, zrows_t)
    ca = c2[:NR].reshape(NP, 1)
    cb = c2[NR:].reshape(NP, 1)

    acc1 = _spmm128(xs, src_p, dst_p, jnp.zeros((RPT, D_IN), f32))
    hs1 = _tc1(acc1[:NP], acc1[NP:], xs, dinv, W1, b1.reshape(1, D_H))

    acc2 = _spmm256(hs1.reshape(2 * NP, D_H // 2), src_p, dst_p,
                    jnp.zeros((RPT, D_H // 2), f32))
    u8 = _tc2(acc2[:NP], acc2[NP:], hs1, dinv, ca, cb,
              W2, b2.reshape(1, D_H))

    Wh = jnp.concatenate(
        [Wl, Wi, Wg, jnp.zeros((D_OUT, 5), f32)], axis=1)
    bh = jnp.concatenate(
        [bl, bi, bg, jnp.zeros((5,), f32)]).reshape(1, 8)
    emb, heads = _tc3(u8[0:1], W3, b3.reshape(1, D_OUT), Wh, bh)

    return (emb, heads[:, 0:1], heads[:, 1:2], heads[:, 2:3])
